# baseline (speedup 1.0000x reference)
import jax
import jax.numpy as jnp
from jax import lax
from jax.experimental import pallas as pl
from jax.experimental.pallas import tpu as pltpu
from jax.experimental.pallas import tpu_sc as plsc

B = 16384
D = 32
NC = 2
NS = 16
NW = NC * NS
B_PER_W = B // NW
GCH = 128
NGC = B_PER_W // GCH
L = 16

BLKN = 65536
FBLK = 4096

_MESH = plsc.VectorSubcoreMesh(
    core_axis_name="c", subcore_axis_name="s", num_cores=NC, num_subcores=NS
)


def _matvec_body(tT, w, out):
    out[...] = jnp.sum(tT[...] * w[...], axis=0)


def _score(table, w, blkn):
    n = table.shape[0]
    grid = (n + blkn - 1) // blkn
    return pl.pallas_call(
        _matvec_body,
        grid=(grid,),
        in_specs=[
            pl.BlockSpec((D, blkn), lambda i: (0, i)),
            pl.BlockSpec((D, 1), lambda i: (0, 0)),
        ],
        out_specs=pl.BlockSpec((blkn,), lambda i: (i,)),
        out_shape=jax.ShapeDtypeStruct((grid * blkn,), jnp.float32),
    )(table.T, w)


def _sc_gather_body(idx_hbm, s_hbm, g_hbm, idx_v, val_v, sem):
    wid = lax.axis_index("s") * NC + lax.axis_index("c")
    base = wid * B_PER_W
    pltpu.sync_copy(idx_hbm.at[pl.ds(base, B_PER_W)], idx_v)
    copies = []
    for j in range(NGC):
        copies.append(
            pltpu.async_copy(
                s_hbm.at[idx_v.at[pl.ds(j * GCH, GCH)]],
                val_v.at[pl.ds(j * GCH, GCH)],
                sem,
            )
        )
    for c in copies:
        c.wait()
    pltpu.sync_copy(val_v, g_hbm.at[pl.ds(base, B_PER_W)])


def _sc_gather(idx, scores):
    return pl.kernel(
        _sc_gather_body,
        out_type=jax.ShapeDtypeStruct((B,), jnp.float32),
        mesh=_MESH,
        scratch_types=[
            pltpu.VMEM((B_PER_W,), jnp.int32),
            pltpu.VMEM((B_PER_W,), jnp.float32),
            pltpu.SemaphoreType.DMA,
        ],
    )(idx, scores)


def _sc_gather_final_body(idx_hbm, s_hbm, gm_hbm, d_hbm, out_hbm,
                          idx_v, val_v, gm_v, d_v, sem, sem2):
    wid = lax.axis_index("s") * NC + lax.axis_index("c")
    base = wid * B_PER_W
    pltpu.sync_copy(idx_hbm.at[pl.ds(base, B_PER_W)], idx_v)
    copies = []
    for j in range(NGC):
        copies.append(
            pltpu.async_copy(
                s_hbm.at[idx_v.at[pl.ds(j * GCH, GCH)]],
                val_v.at[pl.ds(j * GCH, GCH)],
                sem,
            )
        )
    c_gm = pltpu.async_copy(gm_hbm.at[pl.ds(base, B_PER_W)], gm_v, sem2)
    c_d = pltpu.async_copy(d_hbm.at[pl.ds(base, B_PER_W)], d_v, sem2)
    for c in copies:
        c.wait()
    c_gm.wait()
    c_d.wait()
    for k in range(B_PER_W // L):
        s = pl.ds(k * L, L)
        val_v[s] = val_v[s] + gm_v[s] + d_v[s]
    pltpu.sync_copy(val_v, out_hbm.at[pl.ds(base, B_PER_W)])


def _sc_gather_final(idx, scores, gm, d):
    return pl.kernel(
        _sc_gather_final_body,
        out_type=jax.ShapeDtypeStruct((B,), jnp.float32),
        mesh=_MESH,
        scratch_types=[
            pltpu.VMEM((B_PER_W,), jnp.int32),
            pltpu.VMEM((B_PER_W,), jnp.float32),
            pltpu.VMEM((B_PER_W,), jnp.float32),
            pltpu.VMEM((B_PER_W,), jnp.float32),
            pltpu.SemaphoreType.DMA,
            pltpu.SemaphoreType.DMA,
        ],
    )(idx, scores, gm, d)


def _mlp_body(othT, w_h, b_h, w_o, b_o, out):
    z = lax.dot_general(
        othT[...], w_h[...], (((0,), (0,)), ((), ())),
        preferred_element_type=jnp.float32,
    ) + b_h[...]
    a = jnp.where(z >= 0, z, 0.01 * z)
    d = jnp.dot(a, w_o[2 * D:, :], preferred_element_type=jnp.float32)
    out[...] = d[:, 0] + b_o[...]


def _mlp(others_inp, W_h, b_h, W_o, b_o):
    return pl.pallas_call(
        _mlp_body,
        grid=(B // FBLK,),
        in_specs=[
            pl.BlockSpec((64, FBLK), lambda i: (0, i)),
            pl.BlockSpec((64, 64), lambda i: (0, 0)),
            pl.BlockSpec((64,), lambda i: (0,)),
            pl.BlockSpec((128, 1), lambda i: (0, 0)),
            pl.BlockSpec((1,), lambda i: (0,)),
        ],
        out_specs=pl.BlockSpec((FBLK,), lambda i: (i,)),
        out_shape=jax.ShapeDtypeStruct((B,), jnp.float32),
    )(others_inp.T, W_h, b_h, W_o, b_o)


def kernel(user_inp, movie_inp, others_inp, user_table, movie_table, W_h, b_h, W_o, b_o):
    uin = user_inp.astype(jnp.int32)
    min_ = movie_inp.astype(jnp.int32)
    d = _mlp(others_inp, W_h, b_h, W_o, b_o)
    sm = _score(movie_table, W_o[D:2 * D, :], 16384)
    gm = _sc_gather(min_, sm)
    su = _score(user_table, W_o[0:D, :], BLKN)
    out = _sc_gather_final(uin, su, gm, d)
    return out


# speedup vs baseline: 1.1565x; 1.1565x over previous
import jax
import jax.numpy as jnp
from jax import lax
from jax.experimental import pallas as pl
from jax.experimental.pallas import tpu as pltpu
from jax.experimental.pallas import tpu_sc as plsc

B = 16384
D = 32
NC = 2
NS = 16
NW = NC * NS
B_PER_W = B // NW
GCH = 128
NGC = B_PER_W // GCH
L = 16

BLKN = 65536
FBLK = 4096

_MESH = plsc.VectorSubcoreMesh(
    core_axis_name="c", subcore_axis_name="s", num_cores=NC, num_subcores=NS
)


def _matvec_body(tT, wT, out):
    r = jnp.dot(wT[...], tT[...], preferred_element_type=jnp.float32)
    out[...] = r[0, :]


def _score(table, w, blkn):
    n = table.shape[0]
    grid = (n + blkn - 1) // blkn
    return pl.pallas_call(
        _matvec_body,
        grid=(grid,),
        in_specs=[
            pl.BlockSpec((D, blkn), lambda i: (0, i)),
            pl.BlockSpec((1, D), lambda i: (0, 0)),
        ],
        out_specs=pl.BlockSpec((blkn,), lambda i: (i,)),
        out_shape=jax.ShapeDtypeStruct((grid * blkn,), jnp.float32),
    )(table.T, w.T)


def _sc_gather_body(idx_hbm, s_hbm, g_hbm, idx_v, val_v, sem):
    wid = lax.axis_index("s") * NC + lax.axis_index("c")
    base = wid * B_PER_W
    pltpu.sync_copy(idx_hbm.at[pl.ds(base, B_PER_W)], idx_v)
    copies = []
    for j in range(NGC):
        copies.append(
            pltpu.async_copy(
                s_hbm.at[idx_v.at[pl.ds(j * GCH, GCH)]],
                val_v.at[pl.ds(j * GCH, GCH)],
                sem,
            )
        )
    for c in copies:
        c.wait()
    pltpu.sync_copy(val_v, g_hbm.at[pl.ds(base, B_PER_W)])


def _sc_gather(idx, scores):
    return pl.kernel(
        _sc_gather_body,
        out_type=jax.ShapeDtypeStruct((B,), jnp.float32),
        mesh=_MESH,
        scratch_types=[
            pltpu.VMEM((B_PER_W,), jnp.int32),
            pltpu.VMEM((B_PER_W,), jnp.float32),
            pltpu.SemaphoreType.DMA,
        ],
    )(idx, scores)


def _sc_gather_final_body(idx_hbm, s_hbm, gm_hbm, d_hbm, out_hbm,
                          idx_v, val_v, gm_v, d_v, sem, sem2):
    wid = lax.axis_index("s") * NC + lax.axis_index("c")
    base = wid * B_PER_W
    pltpu.sync_copy(idx_hbm.at[pl.ds(base, B_PER_W)], idx_v)
    copies = []
    for j in range(NGC):
        copies.append(
            pltpu.async_copy(
                s_hbm.at[idx_v.at[pl.ds(j * GCH, GCH)]],
                val_v.at[pl.ds(j * GCH, GCH)],
                sem,
            )
        )
    c_gm = pltpu.async_copy(gm_hbm.at[pl.ds(base, B_PER_W)], gm_v, sem2)
    c_d = pltpu.async_copy(d_hbm.at[pl.ds(base, B_PER_W)], d_v, sem2)
    for c in copies:
        c.wait()
    c_gm.wait()
    c_d.wait()
    for k in range(B_PER_W // L):
        s = pl.ds(k * L, L)
        val_v[s] = val_v[s] + gm_v[s] + d_v[s]
    pltpu.sync_copy(val_v, out_hbm.at[pl.ds(base, B_PER_W)])


def _sc_gather_final(idx, scores, gm, d):
    return pl.kernel(
        _sc_gather_final_body,
        out_type=jax.ShapeDtypeStruct((B,), jnp.float32),
        mesh=_MESH,
        scratch_types=[
            pltpu.VMEM((B_PER_W,), jnp.int32),
            pltpu.VMEM((B_PER_W,), jnp.float32),
            pltpu.VMEM((B_PER_W,), jnp.float32),
            pltpu.VMEM((B_PER_W,), jnp.float32),
            pltpu.SemaphoreType.DMA,
            pltpu.SemaphoreType.DMA,
        ],
    )(idx, scores, gm, d)


def _mlp_body(othT, w_hT, b_h2, w_o3T, b_o, out):
    zT = jnp.dot(w_hT[...], othT[...], preferred_element_type=jnp.float32)
    zT = zT + b_h2[...]
    a = jnp.maximum(zT, 0.01 * zT)
    d = jnp.dot(w_o3T[...], a, preferred_element_type=jnp.float32)
    out[...] = d[0, :] + b_o[...]


def _mlp(others_inp, W_h, b_h, W_o, b_o):
    return pl.pallas_call(
        _mlp_body,
        grid=(B // FBLK,),
        in_specs=[
            pl.BlockSpec((64, FBLK), lambda i: (0, i)),
            pl.BlockSpec((64, 64), lambda i: (0, 0)),
            pl.BlockSpec((64, 1), lambda i: (0, 0)),
            pl.BlockSpec((1, 64), lambda i: (0, 0)),
            pl.BlockSpec((1,), lambda i: (0,)),
        ],
        out_specs=pl.BlockSpec((FBLK,), lambda i: (i,)),
        out_shape=jax.ShapeDtypeStruct((B,), jnp.float32),
    )(others_inp.T, W_h.T, b_h.reshape(64, 1), W_o[2 * D:, :].T, b_o)


def kernel(user_inp, movie_inp, others_inp, user_table, movie_table, W_h, b_h, W_o, b_o):
    uin = user_inp.astype(jnp.int32)
    min_ = movie_inp.astype(jnp.int32)
    d = _mlp(others_inp, W_h, b_h, W_o, b_o)
    sm = _score(movie_table, W_o[D:2 * D, :], 16384)
    gm = _sc_gather(min_, sm)
    su = _score(user_table, W_o[0:D, :], BLKN)
    out = _sc_gather_final(uin, su, gm, d)
    return out
